# Initial kernel scaffold; baseline (speedup 1.0000x reference)
import jax
import jax.numpy as jnp
from jax.experimental import pallas as pl


def kernel(x, edge_index, edge_attr, smiles, batch, W1, b1, W2, b2, W3, b3):
    raise NotImplementedError("write your pallas kernel here")


# speedup vs baseline: 7.0912x; 7.0912x over previous
import functools

import jax
import jax.numpy as jnp
from jax import lax
from jax.experimental import pallas as pl
from jax.experimental.pallas import tpu as pltpu
from jax.experimental.pallas import tpu_sc as plsc

_N = 10000
_E = 320000
_F = 128
_FE = 16
_G = 256
_NC = 2
_NS = 16
_NW = _NC * _NS
_EPW = _E // _NW
_QBLK = 5000
_QGRID = _E // _QBLK
_L = 16


def _prep_body(x_ref, ea_ref, w1t_ref, b1c_ref, w2t_ref, w3t_ref, p_ref, q_ref):
    i = pl.program_id(0)
    vt = jnp.dot(w2t_ref[...], w3t_ref[...], preferred_element_type=jnp.float32, precision=lax.Precision.HIGHEST)
    ut = jnp.dot(w1t_ref[...], vt, preferred_element_type=jnp.float32, precision=lax.Precision.HIGHEST)
    c1 = jnp.sum(vt * b1c_ref[...])
    u_et = ut[_F:, :]
    q_ref[...] = jnp.dot(
        ea_ref[...], u_et, preferred_element_type=jnp.float32, precision=lax.Precision.HIGHEST) + c1

    @pl.when(i == 0)
    def _():
        u_xt = ut[:_F, :]
        p_ref[...] = jnp.dot(
            x_ref[...], u_xt, preferred_element_type=jnp.float32, precision=lax.Precision.HIGHEST)


def _prep(x, ea, w1t, b1c, w2t, w3t):
    return pl.pallas_call(
        _prep_body,
        grid=(_QGRID,),
        in_specs=[
            pl.BlockSpec((_N, _F), lambda i: (0, 0)),
            pl.BlockSpec((_QBLK, _FE), lambda i: (i, 0)),
            pl.BlockSpec(w1t.shape, lambda i: (0, 0)),
            pl.BlockSpec(b1c.shape, lambda i: (0, 0)),
            pl.BlockSpec(w2t.shape, lambda i: (0, 0)),
            pl.BlockSpec(w3t.shape, lambda i: (0, 0)),
        ],
        out_specs=[
            pl.BlockSpec((_N, 1), lambda i: (0, 0)),
            pl.BlockSpec((_QBLK, 1), lambda i: (i, 0)),
        ],
        out_shape=[
            jax.ShapeDtypeStruct((_N, 1), jnp.float32),
            jax.ShapeDtypeStruct((_E, 1), jnp.float32),
        ],
    )(x, ea, w1t, b1c, w2t, w3t)


def _sc_edges_body(dst_hbm, src_hbm, q_hbm, batch_hbm, p_hbm, out_hbm,
                   dst_v, src_v, q_v, batch_v, p_v, acc_v):
    wid = lax.axis_index("s") * _NC + lax.axis_index("c")
    base = wid * _EPW
    pltpu.sync_copy(batch_hbm, batch_v)
    pltpu.sync_copy(p_hbm, p_v)
    pltpu.sync_copy(dst_hbm.at[pl.ds(base, _EPW)], dst_v)
    pltpu.sync_copy(src_hbm.at[pl.ds(base, _EPW)], src_v)
    pltpu.sync_copy(q_hbm.at[pl.ds(base, _EPW)], q_v)

    zeros = jnp.zeros((_L,), jnp.float32)

    def zero_body(i, carry):
        acc_v[pl.ds(i * _L, _L)] = zeros
        return carry

    lax.fori_loop(0, _G * _L // _L, zero_body, 0)

    lane = lax.iota(jnp.int32, _L)

    def body(i, carry):
        d16 = dst_v[pl.ds(i * _L, _L)]
        s16 = src_v[pl.ds(i * _L, _L)]
        q16 = q_v[pl.ds(i * _L, _L)]
        g16 = plsc.load_gather(batch_v, [d16])
        p16 = plsc.load_gather(p_v, [s16])
        idx = g16 * _L + lane
        plsc.addupdate_scatter(acc_v, [idx], p16 + q16)
        return carry

    lax.fori_loop(0, _EPW // _L, body, 0)
    pltpu.sync_copy(acc_v, out_hbm.at[wid])


def _sc_edges(dst, src, q, batch, p):
    mesh = plsc.VectorSubcoreMesh(core_axis_name="c", subcore_axis_name="s")
    run = pl.kernel(
        _sc_edges_body, mesh=mesh,
        compiler_params=pltpu.CompilerParams(needs_layout_passes=False),
        out_type=jax.ShapeDtypeStruct((_NW, _G * _L), jnp.float32),
        scratch_types=[
            pltpu.VMEM((_EPW,), jnp.int32),
            pltpu.VMEM((_EPW,), jnp.int32),
            pltpu.VMEM((_EPW,), jnp.float32),
            pltpu.VMEM((_N,), jnp.int32),
            pltpu.VMEM((_N,), jnp.float32),
            pltpu.VMEM((_G * _L,), jnp.float32),
        ],
    )
    return run(dst, src, q, batch, p)


def _finish_body(part_ref, w3_ref, b2_ref, b3_ref, o_ref):
    s = jnp.sum(part_ref[...], axis=0)
    s2 = jnp.sum(s, axis=1, keepdims=True)
    c0 = jnp.sum(w3_ref[...] * b2_ref[...]) + b3_ref[0, 0]
    o_ref[...] = s2 + c0


def _finish(part3, w3, b2r, b3r):
    return pl.pallas_call(
        _finish_body,
        out_shape=jax.ShapeDtypeStruct((_G, 1), jnp.float32),
    )(part3, w3, b2r, b3r)


def kernel(x, edge_index, edge_attr, smiles, batch, W1, b1, W2, b2, W3, b3):
    src = edge_index[0]
    dst = edge_index[1]
    p, q = _prep(x, edge_attr, W1.T, b1.reshape(-1, 1), W2.T, W3.T)
    part = _sc_edges(dst, src, q.reshape(_E), batch, p.reshape(_N))
    part3 = part.reshape(_NW, _G, _L)
    return _finish(part3, W3, b2.reshape(1, -1), b3.reshape(1, 1))


# speedup vs baseline: 34.8443x; 4.9137x over previous
import functools

import jax
import jax.numpy as jnp
from jax import lax
from jax.experimental import pallas as pl
from jax.experimental.pallas import tpu as pltpu
from jax.experimental.pallas import tpu_sc as plsc

_N = 10000
_E = 320000
_F = 128
_FE = 16
_G = 256
_NC = 2
_NS = 16
_NW = _NC * _NS
_EPW = _E // _NW
_QBLK = 32000
_QGRID = _E // _QBLK
_L = 16


def _prep_body(xt_ref, eat_ref, w1_ref, b1r_ref, w2_ref, w3_ref, p_ref, q_ref):
    i = pl.program_id(0)
    v = jnp.dot(w3_ref[...], w2_ref[...], preferred_element_type=jnp.float32,
                precision=lax.Precision.HIGHEST)
    u = jnp.dot(v, w1_ref[...], preferred_element_type=jnp.float32,
                precision=lax.Precision.HIGHEST)
    c1 = jnp.sum(v * b1r_ref[...])
    u_e = u[:, _F:]
    q_ref[...] = jnp.dot(
        u_e, eat_ref[...], preferred_element_type=jnp.float32,
        precision=lax.Precision.HIGHEST) + c1

    @pl.when(i == 0)
    def _():
        u_x = u[:, :_F]
        p_ref[...] = jnp.dot(
            u_x, xt_ref[...], preferred_element_type=jnp.float32,
            precision=lax.Precision.HIGHEST)


def _prep(xt, eat, w1, b1r, w2, w3):
    return pl.pallas_call(
        _prep_body,
        grid=(_QGRID,),
        in_specs=[
            pl.BlockSpec((_F, _N), lambda i: (0, 0)),
            pl.BlockSpec((_FE, _QBLK), lambda i: (0, i)),
            pl.BlockSpec(w1.shape, lambda i: (0, 0)),
            pl.BlockSpec(b1r.shape, lambda i: (0, 0)),
            pl.BlockSpec(w2.shape, lambda i: (0, 0)),
            pl.BlockSpec(w3.shape, lambda i: (0, 0)),
        ],
        out_specs=[
            pl.BlockSpec((1, _N), lambda i: (0, 0)),
            pl.BlockSpec((1, _QBLK), lambda i: (0, i)),
        ],
        out_shape=[
            jax.ShapeDtypeStruct((1, _N), jnp.float32),
            jax.ShapeDtypeStruct((1, _E), jnp.float32),
        ],
    )(xt, eat, w1, b1r, w2, w3)


def _sc_edges_body(dst_hbm, src_hbm, q_hbm, batch_hbm, p_hbm, out_hbm,
                   dst_v, src_v, q_v, batch_v, p_v, acc_v):
    wid = lax.axis_index("s") * _NC + lax.axis_index("c")
    base = wid * _EPW
    pltpu.sync_copy(batch_hbm, batch_v)
    pltpu.sync_copy(p_hbm, p_v)
    pltpu.sync_copy(dst_hbm.at[pl.ds(base, _EPW)], dst_v)
    pltpu.sync_copy(src_hbm.at[pl.ds(base, _EPW)], src_v)
    pltpu.sync_copy(q_hbm.at[pl.ds(base, _EPW)], q_v)

    zeros = jnp.zeros((_L,), jnp.float32)

    def zero_body(i, carry):
        acc_v[pl.ds(i * _L, _L)] = zeros
        return carry

    lax.fori_loop(0, _G * _L // _L, zero_body, 0)

    lane = lax.iota(jnp.int32, _L)

    def body(i, carry):
        d16 = dst_v[pl.ds(i * _L, _L)]
        s16 = src_v[pl.ds(i * _L, _L)]
        q16 = q_v[pl.ds(i * _L, _L)]
        g16 = plsc.load_gather(batch_v, [d16])
        p16 = plsc.load_gather(p_v, [s16])
        idx = g16 * _L + lane
        plsc.addupdate_scatter(acc_v, [idx], p16 + q16)
        return carry

    lax.fori_loop(0, _EPW // _L, body, 0)
    pltpu.sync_copy(acc_v, out_hbm.at[wid])


def _sc_edges(dst, src, q, batch, p):
    mesh = plsc.VectorSubcoreMesh(core_axis_name="c", subcore_axis_name="s")
    run = pl.kernel(
        _sc_edges_body, mesh=mesh,
        compiler_params=pltpu.CompilerParams(needs_layout_passes=False),
        out_type=jax.ShapeDtypeStruct((_NW, _G * _L), jnp.float32),
        scratch_types=[
            pltpu.VMEM((_EPW,), jnp.int32),
            pltpu.VMEM((_EPW,), jnp.int32),
            pltpu.VMEM((_EPW,), jnp.float32),
            pltpu.VMEM((_N,), jnp.int32),
            pltpu.VMEM((_N,), jnp.float32),
            pltpu.VMEM((_G * _L,), jnp.float32),
        ],
    )
    return run(dst, src, q, batch, p)


def _finish_body(part_ref, w3_ref, b2_ref, b3_ref, o_ref):
    colsum = jnp.sum(part_ref[...], axis=0, keepdims=True)
    m = lax.broadcasted_iota(jnp.int32, (_G * _L, _G), 0)
    c = lax.broadcasted_iota(jnp.int32, (_G * _L, _G), 1)
    sel = jnp.where(m // _L == c, 1.0, 0.0).astype(jnp.float32)
    s2 = jnp.dot(colsum, sel, preferred_element_type=jnp.float32,
                 precision=lax.Precision.HIGHEST)
    c0 = jnp.sum(w3_ref[...] * b2_ref[...]) + b3_ref[0, 0]
    o_ref[...] = s2 + c0


def _finish(part, w3, b2r, b3r):
    return pl.pallas_call(
        _finish_body,
        out_shape=jax.ShapeDtypeStruct((1, _G), jnp.float32),
    )(part, w3, b2r, b3r)


def kernel(x, edge_index, edge_attr, smiles, batch, W1, b1, W2, b2, W3, b3):
    src = edge_index[0]
    dst = edge_index[1]
    p, q = _prep(x.T, edge_attr.T, W1, b1.reshape(1, -1), W2, W3)
    part = _sc_edges(dst, src, q.reshape(_E), batch, p.reshape(_N))
    out = _finish(part, W3, b2.reshape(1, -1), b3.reshape(1, 1))
    return out.reshape(_G, 1)


# speedup vs baseline: 47.7404x; 1.3701x over previous
import functools

import jax
import jax.numpy as jnp
from jax import lax
from jax.experimental import pallas as pl
from jax.experimental.pallas import tpu as pltpu
from jax.experimental.pallas import tpu_sc as plsc

_N = 10000
_E = 320000
_F = 128
_FE = 16
_G = 256
_NC = 2
_NS = 16
_NW = _NC * _NS
_EPW = _E // _NW
_QBLK = 32000
_QGRID = _E // _QBLK
_L = 16
_CP = 10112


def _prep_body(xt_ref, eat_ref, ei_ref, w1_ref, b1r_ref, w2_ref, w3_ref,
               p_ref, q_ref, src_ref, dst_ref):
    i = pl.program_id(0)
    v = jnp.dot(w3_ref[...], w2_ref[...], preferred_element_type=jnp.float32,
                precision=lax.Precision.HIGHEST)
    u = jnp.dot(v, w1_ref[...], preferred_element_type=jnp.float32,
                precision=lax.Precision.HIGHEST)
    c1 = jnp.sum(v * b1r_ref[...])
    u_e = u[:, _F:]
    q_ref[...] = jnp.dot(
        u_e, eat_ref[...], preferred_element_type=jnp.float32,
        precision=lax.Precision.HIGHEST) + c1
    ei = ei_ref[...]
    src_ref[...] = ei[0:1, :]
    dst_ref[...] = ei[1:2, :]

    @pl.when(i == 0)
    def _():
        u_x = u[:, :_F]
        p_ref[...] = jnp.dot(
            u_x, xt_ref[...], preferred_element_type=jnp.float32,
            precision=lax.Precision.HIGHEST)


def _prep(xt, eat, ei, w1, b1r, w2, w3):
    return pl.pallas_call(
        _prep_body,
        grid=(_QGRID,),
        in_specs=[
            pl.BlockSpec((_F, _N), lambda i: (0, 0)),
            pl.BlockSpec((_FE, _QBLK), lambda i: (0, i)),
            pl.BlockSpec((2, _QBLK), lambda i: (0, i)),
            pl.BlockSpec(w1.shape, lambda i: (0, 0)),
            pl.BlockSpec(b1r.shape, lambda i: (0, 0)),
            pl.BlockSpec(w2.shape, lambda i: (0, 0)),
            pl.BlockSpec(w3.shape, lambda i: (0, 0)),
        ],
        out_specs=[
            pl.BlockSpec((1, _N), lambda i: (0, 0)),
            pl.BlockSpec((1, _QBLK), lambda i: (0, i)),
            pl.BlockSpec((1, _QBLK), lambda i: (0, i)),
            pl.BlockSpec((1, _QBLK), lambda i: (0, i)),
        ],
        out_shape=[
            jax.ShapeDtypeStruct((1, _N), jnp.float32),
            jax.ShapeDtypeStruct((1, _E), jnp.float32),
            jax.ShapeDtypeStruct((1, _E), jnp.int32),
            jax.ShapeDtypeStruct((1, _E), jnp.int32),
        ],
    )(xt, eat, ei, w1, b1r, w2, w3)


def _sc_edges_body(src_hbm, dst_hbm, q_hbm, batch_hbm, p_hbm, out_hbm,
                   dst_v, src_v, q_v, batch_v, p_v, acc_v):
    wid = lax.axis_index("s") * _NC + lax.axis_index("c")
    base = wid * _EPW
    abase = pl.multiple_of((base // 128) * 128, 128)
    off0 = base - abase
    pltpu.sync_copy(batch_hbm, batch_v)
    pltpu.sync_copy(p_hbm.at[0], p_v)
    pltpu.sync_copy(dst_hbm.at[0, pl.ds(abase, _CP)], dst_v)
    pltpu.sync_copy(src_hbm.at[0, pl.ds(abase, _CP)], src_v)
    pltpu.sync_copy(q_hbm.at[0, pl.ds(abase, _CP)], q_v)

    zeros = jnp.zeros((_L,), jnp.float32)

    def zero_body(i, carry):
        acc_v[pl.ds(i * _L, _L)] = zeros
        return carry

    lax.fori_loop(0, _G * _L // _L, zero_body, 0)

    lane = lax.iota(jnp.int32, _L)
    _UNROLL = 5

    def body(i, carry):
        for u in range(_UNROLL):
            off = off0 + i * (_L * _UNROLL) + u * _L
            d16 = dst_v[pl.ds(off, _L)]
            s16 = src_v[pl.ds(off, _L)]
            q16 = q_v[pl.ds(off, _L)]
            g16 = plsc.load_gather(batch_v, [d16])
            p16 = plsc.load_gather(p_v, [s16])
            idx = g16 * _L + lane
            plsc.addupdate_scatter(acc_v, [idx], p16 + q16)
        return carry

    lax.fori_loop(0, _EPW // (_L * _UNROLL), body, 0)
    pltpu.sync_copy(acc_v, out_hbm.at[wid])


def _sc_edges(src2, dst2, q2, batch, p2):
    mesh = plsc.VectorSubcoreMesh(core_axis_name="c", subcore_axis_name="s")
    run = pl.kernel(
        _sc_edges_body, mesh=mesh,
        compiler_params=pltpu.CompilerParams(needs_layout_passes=False),
        out_type=jax.ShapeDtypeStruct((_NW, _G * _L), jnp.float32),
        scratch_types=[
            pltpu.VMEM((_CP,), jnp.int32),
            pltpu.VMEM((_CP,), jnp.int32),
            pltpu.VMEM((_CP,), jnp.float32),
            pltpu.VMEM((_N,), jnp.int32),
            pltpu.VMEM((_N,), jnp.float32),
            pltpu.VMEM((_G * _L,), jnp.float32),
        ],
    )
    return run(src2, dst2, q2, batch, p2)


def _finish_body(part_ref, w3_ref, b2_ref, b3_ref, o_ref):
    colsum = jnp.sum(part_ref[...], axis=0, keepdims=True)
    m = lax.broadcasted_iota(jnp.int32, (_G * _L, _G), 0)
    c = lax.broadcasted_iota(jnp.int32, (_G * _L, _G), 1)
    sel = jnp.where(m // _L == c, 1.0, 0.0).astype(jnp.float32)
    s2 = jnp.dot(colsum, sel, preferred_element_type=jnp.float32,
                 precision=lax.Precision.HIGHEST)
    c0 = jnp.sum(w3_ref[...] * b2_ref[...]) + b3_ref[0, 0]
    o_ref[...] = s2 + c0


def _finish(part, w3, b2r, b3r):
    return pl.pallas_call(
        _finish_body,
        out_shape=jax.ShapeDtypeStruct((1, _G), jnp.float32),
    )(part, w3, b2r, b3r)


def kernel(x, edge_index, edge_attr, smiles, batch, W1, b1, W2, b2, W3, b3):
    p, q, src2, dst2 = _prep(x.T, edge_attr.T, edge_index, W1,
                             b1.reshape(1, -1), W2, W3)
    part = _sc_edges(src2, dst2, q, batch, p)
    out = _finish(part, W3, b2.reshape(1, -1), b3.reshape(1, 1))
    return out.reshape(_G, 1)


# speedup vs baseline: 74.9156x; 1.5692x over previous
import functools

import jax
import jax.numpy as jnp
from jax import lax
from jax.experimental import pallas as pl
from jax.experimental.pallas import tpu as pltpu
from jax.experimental.pallas import tpu_sc as plsc

_N = 10000
_E = 320000
_F = 128
_FE = 16
_G = 256
_NC = 2
_NS = 16
_NW = _NC * _NS
_EPW = _E // _NW
_QBLK = 32000
_QGRID = _E // _QBLK
_L = 16
_CP = 10112


def _pfold_body(x_ref, w1_ref, w2_ref, w3_ref, p_ref):
    v = jnp.dot(w3_ref[...], w2_ref[...], preferred_element_type=jnp.float32,
                precision=lax.Precision.HIGHEST)
    u = jnp.dot(v, w1_ref[...], preferred_element_type=jnp.float32,
                precision=lax.Precision.HIGHEST)
    u_x = u[:, :_F]
    p_ref[...] = lax.dot_general(
        u_x, x_ref[...], (((1,), (1,)), ((), ())),
        preferred_element_type=jnp.float32)


def _pfold(x, w1, w2, w3):
    return pl.pallas_call(
        _pfold_body,
        out_shape=jax.ShapeDtypeStruct((1, _N), jnp.float32),
    )(x, w1, w2, w3)


def _prep_body(eat_ref, ei_ref, w1_ref, b1r_ref, w2_ref, w3_ref,
               q_ref, src_ref, dst_ref):
    v = jnp.dot(w3_ref[...], w2_ref[...], preferred_element_type=jnp.float32,
                precision=lax.Precision.HIGHEST)
    u = jnp.dot(v, w1_ref[...], preferred_element_type=jnp.float32,
                precision=lax.Precision.HIGHEST)
    c1 = jnp.sum(v * b1r_ref[...])
    u_e = u[:, _F:]
    q_ref[...] = jnp.dot(
        u_e, eat_ref[...], preferred_element_type=jnp.float32) + c1
    ei = ei_ref[...]
    src_ref[...] = ei[0:1, :]
    dst_ref[...] = ei[1:2, :]


def _prep(eat, ei, w1, b1r, w2, w3):
    return pl.pallas_call(
        _prep_body,
        grid=(_QGRID,),
        in_specs=[
            pl.BlockSpec((_FE, _QBLK), lambda i: (0, i)),
            pl.BlockSpec((2, _QBLK), lambda i: (0, i)),
            pl.BlockSpec(w1.shape, lambda i: (0, 0)),
            pl.BlockSpec(b1r.shape, lambda i: (0, 0)),
            pl.BlockSpec(w2.shape, lambda i: (0, 0)),
            pl.BlockSpec(w3.shape, lambda i: (0, 0)),
        ],
        out_specs=[
            pl.BlockSpec((1, _QBLK), lambda i: (0, i)),
            pl.BlockSpec((1, _QBLK), lambda i: (0, i)),
            pl.BlockSpec((1, _QBLK), lambda i: (0, i)),
        ],
        out_shape=[
            jax.ShapeDtypeStruct((1, _E), jnp.float32),
            jax.ShapeDtypeStruct((1, _E), jnp.int32),
            jax.ShapeDtypeStruct((1, _E), jnp.int32),
        ],
    )(eat, ei, w1, b1r, w2, w3)


def _sc_edges_body(src_hbm, dst_hbm, q_hbm, batch_hbm, p_hbm, out_hbm,
                   dst_v, src_v, q_v, batch_v, p_v, acc_v,
                   sem0, sem1, sem2, sem3, sem4):
    wid = lax.axis_index("s") * _NC + lax.axis_index("c")
    base = wid * _EPW
    abase = pl.multiple_of((base // 128) * 128, 128)
    off0 = base - abase
    c0 = pltpu.async_copy(batch_hbm, batch_v, sem0)
    c1 = pltpu.async_copy(p_hbm.at[0], p_v, sem1)
    c2 = pltpu.async_copy(dst_hbm.at[0, pl.ds(abase, _CP)], dst_v, sem2)
    c3 = pltpu.async_copy(src_hbm.at[0, pl.ds(abase, _CP)], src_v, sem3)
    c4 = pltpu.async_copy(q_hbm.at[0, pl.ds(abase, _CP)], q_v, sem4)

    zeros = jnp.zeros((_L,), jnp.float32)

    @plsc.parallel_loop(0, _G, 1, unroll=8)
    def _(i):
        acc_v[pl.ds(i * _L, _L)] = zeros

    c0.wait()
    c1.wait()
    c2.wait()
    c3.wait()
    c4.wait()

    lane = lax.iota(jnp.int32, _L)

    @plsc.parallel_loop(0, _EPW // _L, 1, unroll=5)
    def _(i):
        off = off0 + i * _L
        d16 = dst_v[pl.ds(off, _L)]
        s16 = src_v[pl.ds(off, _L)]
        q16 = q_v[pl.ds(off, _L)]
        g16 = plsc.load_gather(batch_v, [d16])
        p16 = plsc.load_gather(p_v, [s16])
        idx = g16 * _L + lane
        plsc.addupdate_scatter(acc_v, [idx], p16 + q16)

    pltpu.sync_copy(acc_v, out_hbm.at[wid])


def _sc_edges(src2, dst2, q2, batch, p2):
    mesh = plsc.VectorSubcoreMesh(core_axis_name="c", subcore_axis_name="s")
    run = pl.kernel(
        _sc_edges_body, mesh=mesh,
        compiler_params=pltpu.CompilerParams(needs_layout_passes=False),
        out_type=jax.ShapeDtypeStruct((_NW, _G * _L), jnp.float32),
        scratch_types=[
            pltpu.VMEM((_CP,), jnp.int32),
            pltpu.VMEM((_CP,), jnp.int32),
            pltpu.VMEM((_CP,), jnp.float32),
            pltpu.VMEM((_N,), jnp.int32),
            pltpu.VMEM((_N,), jnp.float32),
            pltpu.VMEM((_G * _L,), jnp.float32),
            pltpu.SemaphoreType.DMA,
            pltpu.SemaphoreType.DMA,
            pltpu.SemaphoreType.DMA,
            pltpu.SemaphoreType.DMA,
            pltpu.SemaphoreType.DMA,
        ],
    )
    return run(src2, dst2, q2, batch, p2)


def _finish_body(part_ref, w3_ref, b2_ref, b3_ref, o_ref):
    colsum = jnp.sum(part_ref[...], axis=0, keepdims=True)
    m = lax.broadcasted_iota(jnp.int32, (_G * _L, _G), 0)
    c = lax.broadcasted_iota(jnp.int32, (_G * _L, _G), 1)
    sel = jnp.where(m // _L == c, 1.0, 0.0).astype(jnp.float32)
    s2 = jnp.dot(colsum, sel, preferred_element_type=jnp.float32,
                 precision=lax.Precision.HIGHEST)
    c0 = jnp.sum(w3_ref[...] * b2_ref[...]) + b3_ref[0, 0]
    o_ref[...] = s2 + c0


def _finish(part, w3, b2r, b3r):
    return pl.pallas_call(
        _finish_body,
        out_shape=jax.ShapeDtypeStruct((1, _G), jnp.float32),
    )(part, w3, b2r, b3r)


def kernel(x, edge_index, edge_attr, smiles, batch, W1, b1, W2, b2, W3, b3):
    p = _pfold(x, W1, W2, W3)
    q, src2, dst2 = _prep(edge_attr.T, edge_index, W1,
                          b1.reshape(1, -1), W2, W3)
    part = _sc_edges(src2, dst2, q, batch, p)
    out = _finish(part, W3, b2.reshape(1, -1), b3.reshape(1, 1))
    return out.reshape(_G, 1)


# speedup vs baseline: 79.7892x; 1.0651x over previous
import functools

import jax
import jax.numpy as jnp
from jax import lax
from jax.experimental import pallas as pl
from jax.experimental.pallas import tpu as pltpu
from jax.experimental.pallas import tpu_sc as plsc

_N = 10000
_E = 320000
_F = 128
_FE = 16
_G = 256
_NC = 2
_NS = 16
_NW = _NC * _NS
_EPW = _E // _NW
_QBLK = 64000
_QGRID = _E // _QBLK
_L = 16
_CP = 10112


def _pfold_body(x_ref, w1_ref, w2_ref, w3_ref, p_ref):
    v = jnp.dot(w3_ref[...], w2_ref[...], preferred_element_type=jnp.float32,
                precision=lax.Precision.HIGHEST)
    u = jnp.dot(v, w1_ref[...], preferred_element_type=jnp.float32,
                precision=lax.Precision.HIGHEST)
    u_x = u[:, :_F]
    p_ref[...] = lax.dot_general(
        u_x, x_ref[...], (((1,), (1,)), ((), ())),
        preferred_element_type=jnp.float32)


def _pfold(x, w1, w2, w3):
    return pl.pallas_call(
        _pfold_body,
        out_shape=jax.ShapeDtypeStruct((1, _N), jnp.float32),
    )(x, w1, w2, w3)


def _prep_body(eat_ref, ei_ref, w1_ref, b1r_ref, w2_ref, w3_ref,
               q_ref, src_ref, dst_ref):
    v = jnp.dot(w3_ref[...], w2_ref[...], preferred_element_type=jnp.float32,
                precision=lax.Precision.HIGHEST)
    u = jnp.dot(v, w1_ref[...], preferred_element_type=jnp.float32,
                precision=lax.Precision.HIGHEST)
    c1 = jnp.sum(v * b1r_ref[...])
    u_e = u[:, _F:]
    q_ref[...] = jnp.dot(
        u_e, eat_ref[...], preferred_element_type=jnp.float32) + c1
    ei = ei_ref[...]
    src_ref[...] = ei[0:1, :]
    dst_ref[...] = ei[1:2, :]


def _prep(eat, ei, w1, b1r, w2, w3):
    return pl.pallas_call(
        _prep_body,
        grid=(_QGRID,),
        in_specs=[
            pl.BlockSpec((_FE, _QBLK), lambda i: (0, i)),
            pl.BlockSpec((2, _QBLK), lambda i: (0, i)),
            pl.BlockSpec(w1.shape, lambda i: (0, 0)),
            pl.BlockSpec(b1r.shape, lambda i: (0, 0)),
            pl.BlockSpec(w2.shape, lambda i: (0, 0)),
            pl.BlockSpec(w3.shape, lambda i: (0, 0)),
        ],
        out_specs=[
            pl.BlockSpec((1, _QBLK), lambda i: (0, i)),
            pl.BlockSpec((1, _QBLK), lambda i: (0, i)),
            pl.BlockSpec((1, _QBLK), lambda i: (0, i)),
        ],
        out_shape=[
            jax.ShapeDtypeStruct((1, _E), jnp.float32),
            jax.ShapeDtypeStruct((1, _E), jnp.int32),
            jax.ShapeDtypeStruct((1, _E), jnp.int32),
        ],
    )(eat, ei, w1, b1r, w2, w3)


def _sc_edges_body(src_hbm, dst_hbm, q_hbm, batch_hbm, p_hbm, out_hbm,
                   dst_v, src_v, q_v, batch_v, p_v, acc_v,
                   sem0, sem1, sem2, sem3, sem4):
    wid = lax.axis_index("s") * _NC + lax.axis_index("c")
    base = wid * _EPW
    abase = pl.multiple_of((base // 128) * 128, 128)
    off0 = base - abase
    c0 = pltpu.async_copy(batch_hbm, batch_v, sem0)
    c1 = pltpu.async_copy(p_hbm.at[0], p_v, sem1)
    c2 = pltpu.async_copy(dst_hbm.at[0, pl.ds(abase, _CP)], dst_v, sem2)
    c3 = pltpu.async_copy(src_hbm.at[0, pl.ds(abase, _CP)], src_v, sem3)
    c4 = pltpu.async_copy(q_hbm.at[0, pl.ds(abase, _CP)], q_v, sem4)

    zeros = jnp.zeros((_L,), jnp.float32)

    @plsc.parallel_loop(0, _G, 1, unroll=8)
    def _(i):
        acc_v[pl.ds(i * _L, _L)] = zeros

    c0.wait()
    c1.wait()
    c2.wait()
    c3.wait()
    c4.wait()

    lane = lax.iota(jnp.int32, _L)

    @plsc.parallel_loop(0, _EPW // _L, 1, unroll=5)
    def _(i):
        off = off0 + i * _L
        d16 = dst_v[pl.ds(off, _L)]
        s16 = src_v[pl.ds(off, _L)]
        q16 = q_v[pl.ds(off, _L)]
        g16 = plsc.load_gather(batch_v, [d16])
        p16 = plsc.load_gather(p_v, [s16])
        idx = g16 * _L + lane
        plsc.addupdate_scatter(acc_v, [idx], p16 + q16)

    pltpu.sync_copy(acc_v, out_hbm.at[wid])


def _sc_edges(src2, dst2, q2, batch, p2):
    mesh = plsc.VectorSubcoreMesh(core_axis_name="c", subcore_axis_name="s")
    run = pl.kernel(
        _sc_edges_body, mesh=mesh,
        compiler_params=pltpu.CompilerParams(needs_layout_passes=False),
        out_type=jax.ShapeDtypeStruct((_NW, _G * _L), jnp.float32),
        scratch_types=[
            pltpu.VMEM((_CP,), jnp.int32),
            pltpu.VMEM((_CP,), jnp.int32),
            pltpu.VMEM((_CP,), jnp.float32),
            pltpu.VMEM((_N,), jnp.int32),
            pltpu.VMEM((_N,), jnp.float32),
            pltpu.VMEM((_G * _L,), jnp.float32),
            pltpu.SemaphoreType.DMA,
            pltpu.SemaphoreType.DMA,
            pltpu.SemaphoreType.DMA,
            pltpu.SemaphoreType.DMA,
            pltpu.SemaphoreType.DMA,
        ],
    )
    return run(src2, dst2, q2, batch, p2)


def _finish_body(part_ref, w3_ref, b2_ref, b3_ref, o_ref):
    colsum = jnp.sum(part_ref[...], axis=0, keepdims=True)
    m = lax.broadcasted_iota(jnp.int32, (_G * _L, _G), 0)
    c = lax.broadcasted_iota(jnp.int32, (_G * _L, _G), 1)
    sel = jnp.where(m // _L == c, 1.0, 0.0).astype(jnp.float32)
    s2 = jnp.dot(colsum, sel, preferred_element_type=jnp.float32,
                 precision=lax.Precision.HIGHEST)
    c0 = jnp.sum(w3_ref[...] * b2_ref[...]) + b3_ref[0, 0]
    o_ref[...] = s2 + c0


def _finish(part, w3, b2r, b3r):
    return pl.pallas_call(
        _finish_body,
        out_shape=jax.ShapeDtypeStruct((1, _G), jnp.float32),
    )(part, w3, b2r, b3r)


def kernel(x, edge_index, edge_attr, smiles, batch, W1, b1, W2, b2, W3, b3):
    p = _pfold(x, W1, W2, W3)
    q, src2, dst2 = _prep(edge_attr.T, edge_index, W1,
                          b1.reshape(1, -1), W2, W3)
    part = _sc_edges(src2, dst2, q, batch, p)
    out = _finish(part, W3, b2.reshape(1, -1), b3.reshape(1, 1))
    return out.reshape(_G, 1)


# speedup vs baseline: 81.6915x; 1.0238x over previous
import functools

import jax
import jax.numpy as jnp
from jax import lax
from jax.experimental import pallas as pl
from jax.experimental.pallas import tpu as pltpu
from jax.experimental.pallas import tpu_sc as plsc

_N = 10000
_E = 320000
_F = 128
_FE = 16
_G = 256
_NC = 2
_NS = 16
_NW = _NC * _NS
_EPW = _E // _NW
_QBLK = 80000
_QGRID = _E // _QBLK
_L = 16
_CP = 10112


def _pfold_body(x_ref, w1_ref, w2_ref, w3_ref, p_ref):
    v = jnp.dot(w3_ref[...], w2_ref[...], preferred_element_type=jnp.float32,
                precision=lax.Precision.HIGHEST)
    u = jnp.dot(v, w1_ref[...], preferred_element_type=jnp.float32,
                precision=lax.Precision.HIGHEST)
    u_x = u[:, :_F]
    p_ref[...] = lax.dot_general(
        u_x, x_ref[...], (((1,), (1,)), ((), ())),
        preferred_element_type=jnp.float32)


def _pfold(x, w1, w2, w3):
    return pl.pallas_call(
        _pfold_body,
        out_shape=jax.ShapeDtypeStruct((1, _N), jnp.float32),
    )(x, w1, w2, w3)


def _prep_body(eat_ref, ei_ref, w1_ref, b1r_ref, w2_ref, w3_ref,
               q_ref, src_ref, dst_ref):
    v = jnp.dot(w3_ref[...], w2_ref[...], preferred_element_type=jnp.float32,
                precision=lax.Precision.HIGHEST)
    u = jnp.dot(v, w1_ref[...], preferred_element_type=jnp.float32,
                precision=lax.Precision.HIGHEST)
    c1 = jnp.sum(v * b1r_ref[...])
    u_e = u[:, _F:]
    q_ref[...] = jnp.dot(
        u_e, eat_ref[...], preferred_element_type=jnp.float32) + c1
    ei = ei_ref[...]
    src_ref[...] = ei[0:1, :]
    dst_ref[...] = ei[1:2, :]


def _prep(eat, ei, w1, b1r, w2, w3):
    return pl.pallas_call(
        _prep_body,
        grid=(_QGRID,),
        in_specs=[
            pl.BlockSpec((_FE, _QBLK), lambda i: (0, i)),
            pl.BlockSpec((2, _QBLK), lambda i: (0, i)),
            pl.BlockSpec(w1.shape, lambda i: (0, 0)),
            pl.BlockSpec(b1r.shape, lambda i: (0, 0)),
            pl.BlockSpec(w2.shape, lambda i: (0, 0)),
            pl.BlockSpec(w3.shape, lambda i: (0, 0)),
        ],
        out_specs=[
            pl.BlockSpec((1, _QBLK), lambda i: (0, i)),
            pl.BlockSpec((1, _QBLK), lambda i: (0, i)),
            pl.BlockSpec((1, _QBLK), lambda i: (0, i)),
        ],
        out_shape=[
            jax.ShapeDtypeStruct((1, _E), jnp.float32),
            jax.ShapeDtypeStruct((1, _E), jnp.int32),
            jax.ShapeDtypeStruct((1, _E), jnp.int32),
        ],
    )(eat, ei, w1, b1r, w2, w3)


def _sc_edges_body(src_hbm, dst_hbm, q_hbm, batch_hbm, p_hbm, out_hbm,
                   dst_v, src_v, q_v, batch_v, p_v, acc_v,
                   sem0, sem1, sem2, sem3, sem4):
    wid = lax.axis_index("s") * _NC + lax.axis_index("c")
    base = wid * _EPW
    abase = pl.multiple_of((base // 128) * 128, 128)
    off0 = base - abase
    c0 = pltpu.async_copy(batch_hbm, batch_v, sem0)
    c1 = pltpu.async_copy(p_hbm.at[0], p_v, sem1)
    c2 = pltpu.async_copy(dst_hbm.at[0, pl.ds(abase, _CP)], dst_v, sem2)
    c3 = pltpu.async_copy(src_hbm.at[0, pl.ds(abase, _CP)], src_v, sem3)
    c4 = pltpu.async_copy(q_hbm.at[0, pl.ds(abase, _CP)], q_v, sem4)

    zeros = jnp.zeros((_L,), jnp.float32)

    @plsc.parallel_loop(0, _G, 1, unroll=8)
    def _(i):
        acc_v[pl.ds(i * _L, _L)] = zeros

    c0.wait()
    c1.wait()
    c2.wait()
    c3.wait()
    c4.wait()

    lane = lax.iota(jnp.int32, _L)

    @plsc.parallel_loop(0, _EPW // _L, 1, unroll=25)
    def _(i):
        off = off0 + i * _L
        d16 = dst_v[pl.ds(off, _L)]
        s16 = src_v[pl.ds(off, _L)]
        q16 = q_v[pl.ds(off, _L)]
        g16 = plsc.load_gather(batch_v, [d16])
        p16 = plsc.load_gather(p_v, [s16])
        idx = g16 * _L + lane
        plsc.addupdate_scatter(acc_v, [idx], p16 + q16)

    pltpu.sync_copy(acc_v, out_hbm.at[wid])


def _sc_edges(src2, dst2, q2, batch, p2):
    mesh = plsc.VectorSubcoreMesh(core_axis_name="c", subcore_axis_name="s")
    run = pl.kernel(
        _sc_edges_body, mesh=mesh,
        compiler_params=pltpu.CompilerParams(needs_layout_passes=False),
        out_type=jax.ShapeDtypeStruct((_NW, _G * _L), jnp.float32),
        scratch_types=[
            pltpu.VMEM((_CP,), jnp.int32),
            pltpu.VMEM((_CP,), jnp.int32),
            pltpu.VMEM((_CP,), jnp.float32),
            pltpu.VMEM((_N,), jnp.int32),
            pltpu.VMEM((_N,), jnp.float32),
            pltpu.VMEM((_G * _L,), jnp.float32),
            pltpu.SemaphoreType.DMA,
            pltpu.SemaphoreType.DMA,
            pltpu.SemaphoreType.DMA,
            pltpu.SemaphoreType.DMA,
            pltpu.SemaphoreType.DMA,
        ],
    )
    return run(src2, dst2, q2, batch, p2)


def _finish_body(part_ref, w3_ref, b2_ref, b3_ref, o_ref):
    colsum = jnp.sum(part_ref[...], axis=0, keepdims=True)
    m = lax.broadcasted_iota(jnp.int32, (_G * _L, _G), 0)
    c = lax.broadcasted_iota(jnp.int32, (_G * _L, _G), 1)
    sel = jnp.where(m // _L == c, 1.0, 0.0).astype(jnp.float32)
    s2 = jnp.dot(colsum, sel, preferred_element_type=jnp.float32,
                 precision=lax.Precision.HIGHEST)
    c0 = jnp.sum(w3_ref[...] * b2_ref[...]) + b3_ref[0, 0]
    o_ref[...] = s2 + c0


def _finish(part, w3, b2r, b3r):
    return pl.pallas_call(
        _finish_body,
        out_shape=jax.ShapeDtypeStruct((1, _G), jnp.float32),
    )(part, w3, b2r, b3r)


def kernel(x, edge_index, edge_attr, smiles, batch, W1, b1, W2, b2, W3, b3):
    p = _pfold(x, W1, W2, W3)
    q, src2, dst2 = _prep(edge_attr.T, edge_index, W1,
                          b1.reshape(1, -1), W2, W3)
    part = _sc_edges(src2, dst2, q, batch, p)
    out = _finish(part, W3, b2.reshape(1, -1), b3.reshape(1, 1))
    return out.reshape(_G, 1)


# speedup vs baseline: 82.7936x; 1.0135x over previous
import functools

import jax
import jax.numpy as jnp
from jax import lax
from jax.experimental import pallas as pl
from jax.experimental.pallas import tpu as pltpu
from jax.experimental.pallas import tpu_sc as plsc

_N = 10000
_E = 320000
_F = 128
_FE = 16
_G = 256
_NC = 2
_NS = 16
_NW = _NC * _NS
_EPW = _E // _NW
_QBLK = 160000
_QGRID = _E // _QBLK
_L = 16
_CP = 10112


def _pfold_body(x_ref, w1_ref, w2_ref, w3_ref, p_ref):
    v = jnp.dot(w3_ref[...], w2_ref[...], preferred_element_type=jnp.float32,
                precision=lax.Precision.HIGHEST)
    u = jnp.dot(v, w1_ref[...], preferred_element_type=jnp.float32,
                precision=lax.Precision.HIGHEST)
    u_x = u[:, :_F]
    p_ref[...] = lax.dot_general(
        u_x, x_ref[...], (((1,), (1,)), ((), ())),
        preferred_element_type=jnp.float32)


def _pfold(x, w1, w2, w3):
    return pl.pallas_call(
        _pfold_body,
        out_shape=jax.ShapeDtypeStruct((1, _N), jnp.float32),
    )(x, w1, w2, w3)


def _prep_body(eat_ref, ei_ref, w1_ref, b1r_ref, w2_ref, w3_ref,
               q_ref, src_ref, dst_ref):
    v = jnp.dot(w3_ref[...], w2_ref[...], preferred_element_type=jnp.float32,
                precision=lax.Precision.HIGHEST)
    u = jnp.dot(v, w1_ref[...], preferred_element_type=jnp.float32,
                precision=lax.Precision.HIGHEST)
    c1 = jnp.sum(v * b1r_ref[...])
    u_e = u[:, _F:]
    q_ref[...] = jnp.dot(
        u_e, eat_ref[...], preferred_element_type=jnp.float32) + c1
    ei = ei_ref[...]
    src_ref[...] = ei[0:1, :]
    dst_ref[...] = ei[1:2, :]


def _prep(eat, ei, w1, b1r, w2, w3):
    return pl.pallas_call(
        _prep_body,
        grid=(_QGRID,),
        in_specs=[
            pl.BlockSpec((_FE, _QBLK), lambda i: (0, i)),
            pl.BlockSpec((2, _QBLK), lambda i: (0, i)),
            pl.BlockSpec(w1.shape, lambda i: (0, 0)),
            pl.BlockSpec(b1r.shape, lambda i: (0, 0)),
            pl.BlockSpec(w2.shape, lambda i: (0, 0)),
            pl.BlockSpec(w3.shape, lambda i: (0, 0)),
        ],
        out_specs=[
            pl.BlockSpec((1, _QBLK), lambda i: (0, i)),
            pl.BlockSpec((1, _QBLK), lambda i: (0, i)),
            pl.BlockSpec((1, _QBLK), lambda i: (0, i)),
        ],
        out_shape=[
            jax.ShapeDtypeStruct((1, _E), jnp.float32),
            jax.ShapeDtypeStruct((1, _E), jnp.int32),
            jax.ShapeDtypeStruct((1, _E), jnp.int32),
        ],
    )(eat, ei, w1, b1r, w2, w3)


def _sc_edges_body(src_hbm, dst_hbm, q_hbm, batch_hbm, p_hbm, out_hbm,
                   dst_v, src_v, q_v, batch_v, p_v, acc_v,
                   sem0, sem1, sem2, sem3, sem4):
    wid = lax.axis_index("s") * _NC + lax.axis_index("c")
    base = wid * _EPW
    abase = pl.multiple_of((base // 128) * 128, 128)
    off0 = base - abase
    c0 = pltpu.async_copy(batch_hbm, batch_v, sem0)
    c1 = pltpu.async_copy(p_hbm.at[0], p_v, sem1)
    c2 = pltpu.async_copy(dst_hbm.at[0, pl.ds(abase, _CP)], dst_v, sem2)
    c3 = pltpu.async_copy(src_hbm.at[0, pl.ds(abase, _CP)], src_v, sem3)
    c4 = pltpu.async_copy(q_hbm.at[0, pl.ds(abase, _CP)], q_v, sem4)

    zeros = jnp.zeros((_L,), jnp.float32)

    @plsc.parallel_loop(0, _G, 1, unroll=8)
    def _(i):
        acc_v[pl.ds(i * _L, _L)] = zeros

    c0.wait()
    c1.wait()
    c2.wait()
    c3.wait()
    c4.wait()

    lane = lax.iota(jnp.int32, _L)

    @plsc.parallel_loop(0, _EPW // _L, 1, unroll=25)
    def _(i):
        off = off0 + i * _L
        d16 = dst_v[pl.ds(off, _L)]
        s16 = src_v[pl.ds(off, _L)]
        q16 = q_v[pl.ds(off, _L)]
        g16 = plsc.load_gather(batch_v, [d16])
        p16 = plsc.load_gather(p_v, [s16])
        idx = g16 * _L + lane
        plsc.addupdate_scatter(acc_v, [idx], p16 + q16)

    pltpu.sync_copy(acc_v, out_hbm.at[wid])


def _sc_edges(src2, dst2, q2, batch, p2):
    mesh = plsc.VectorSubcoreMesh(core_axis_name="c", subcore_axis_name="s")
    run = pl.kernel(
        _sc_edges_body, mesh=mesh,
        compiler_params=pltpu.CompilerParams(needs_layout_passes=False),
        out_type=jax.ShapeDtypeStruct((_NW, _G * _L), jnp.float32),
        scratch_types=[
            pltpu.VMEM((_CP,), jnp.int32),
            pltpu.VMEM((_CP,), jnp.int32),
            pltpu.VMEM((_CP,), jnp.float32),
            pltpu.VMEM((_N,), jnp.int32),
            pltpu.VMEM((_N,), jnp.float32),
            pltpu.VMEM((_G * _L,), jnp.float32),
            pltpu.SemaphoreType.DMA,
            pltpu.SemaphoreType.DMA,
            pltpu.SemaphoreType.DMA,
            pltpu.SemaphoreType.DMA,
            pltpu.SemaphoreType.DMA,
        ],
    )
    return run(src2, dst2, q2, batch, p2)


def _finish_body(part_ref, w3_ref, b2_ref, b3_ref, o_ref):
    colsum = jnp.sum(part_ref[...], axis=0, keepdims=True)
    m = lax.broadcasted_iota(jnp.int32, (_G * _L, _G), 0)
    c = lax.broadcasted_iota(jnp.int32, (_G * _L, _G), 1)
    sel = jnp.where(m // _L == c, 1.0, 0.0).astype(jnp.float32)
    s2 = jnp.dot(colsum, sel, preferred_element_type=jnp.float32,
                 precision=lax.Precision.HIGHEST)
    c0 = jnp.sum(w3_ref[...] * b2_ref[...]) + b3_ref[0, 0]
    o_ref[...] = s2 + c0


def _finish(part, w3, b2r, b3r):
    return pl.pallas_call(
        _finish_body,
        out_shape=jax.ShapeDtypeStruct((1, _G), jnp.float32),
    )(part, w3, b2r, b3r)


def kernel(x, edge_index, edge_attr, smiles, batch, W1, b1, W2, b2, W3, b3):
    p = _pfold(x, W1, W2, W3)
    q, src2, dst2 = _prep(edge_attr.T, edge_index, W1,
                          b1.reshape(1, -1), W2, W3)
    part = _sc_edges(src2, dst2, q, batch, p)
    out = _finish(part, W3, b2.reshape(1, -1), b3.reshape(1, 1))
    return out.reshape(_G, 1)


# speedup vs baseline: 84.9952x; 1.0266x over previous
import functools

import jax
import jax.numpy as jnp
from jax import lax
from jax.experimental import pallas as pl
from jax.experimental.pallas import tpu as pltpu
from jax.experimental.pallas import tpu_sc as plsc

_N = 10000
_E = 320000
_F = 128
_FE = 16
_G = 256
_NC = 2
_NS = 16
_NW = _NC * _NS
_EPW = _E // _NW
_QBLK = 160000
_QGRID = _E // _QBLK
_L = 16
_CP = 10112


def _prep_body(x_ref, eat_ref, ei_ref, w1_ref, b1r_ref, w2_ref, w3_ref,
               p_ref, q_ref, src_ref, dst_ref):
    i = pl.program_id(0)
    v = jnp.dot(w3_ref[...], w2_ref[...], preferred_element_type=jnp.float32,
                precision=lax.Precision.HIGHEST)
    u = jnp.dot(v, w1_ref[...], preferred_element_type=jnp.float32,
                precision=lax.Precision.HIGHEST)
    c1 = jnp.sum(v * b1r_ref[...])
    u_e = u[:, _F:]
    q_ref[...] = jnp.dot(
        u_e, eat_ref[...], preferred_element_type=jnp.float32) + c1
    ei = ei_ref[...]
    src_ref[...] = ei[0:1, :]
    dst_ref[...] = ei[1:2, :]

    @pl.when(i == 0)
    def _():
        u_x = u[:, :_F]
        p_ref[...] = lax.dot_general(
            u_x, x_ref[...], (((1,), (1,)), ((), ())),
            preferred_element_type=jnp.float32)


def _prep(x, eat, ei, w1, b1r, w2, w3):
    return pl.pallas_call(
        _prep_body,
        grid=(_QGRID,),
        in_specs=[
            pl.BlockSpec((_N, _F), lambda i: (0, 0)),
            pl.BlockSpec((_FE, _QBLK), lambda i: (0, i)),
            pl.BlockSpec((2, _QBLK), lambda i: (0, i)),
            pl.BlockSpec(w1.shape, lambda i: (0, 0)),
            pl.BlockSpec(b1r.shape, lambda i: (0, 0)),
            pl.BlockSpec(w2.shape, lambda i: (0, 0)),
            pl.BlockSpec(w3.shape, lambda i: (0, 0)),
        ],
        out_specs=[
            pl.BlockSpec((1, _N), lambda i: (0, 0)),
            pl.BlockSpec((1, _QBLK), lambda i: (0, i)),
            pl.BlockSpec((1, _QBLK), lambda i: (0, i)),
            pl.BlockSpec((1, _QBLK), lambda i: (0, i)),
        ],
        out_shape=[
            jax.ShapeDtypeStruct((1, _N), jnp.float32),
            jax.ShapeDtypeStruct((1, _E), jnp.float32),
            jax.ShapeDtypeStruct((1, _E), jnp.int32),
            jax.ShapeDtypeStruct((1, _E), jnp.int32),
        ],
    )(x, eat, ei, w1, b1r, w2, w3)


def _sc_edges_body(src_hbm, dst_hbm, q_hbm, batch_hbm, p_hbm, out_hbm,
                   dst_v, src_v, q_v, batch_v, p_v, acc_v,
                   sem0, sem1, sem2, sem3, sem4):
    wid = lax.axis_index("s") * _NC + lax.axis_index("c")
    base = wid * _EPW
    abase = pl.multiple_of((base // 128) * 128, 128)
    off0 = base - abase
    c0 = pltpu.async_copy(batch_hbm, batch_v, sem0)
    c1 = pltpu.async_copy(p_hbm.at[0], p_v, sem1)
    c2 = pltpu.async_copy(dst_hbm.at[0, pl.ds(abase, _CP)], dst_v, sem2)
    c3 = pltpu.async_copy(src_hbm.at[0, pl.ds(abase, _CP)], src_v, sem3)
    c4 = pltpu.async_copy(q_hbm.at[0, pl.ds(abase, _CP)], q_v, sem4)

    zeros = jnp.zeros((_L,), jnp.float32)

    @plsc.parallel_loop(0, _G, 1, unroll=8)
    def _(i):
        acc_v[pl.ds(i * _L, _L)] = zeros

    c0.wait()
    c1.wait()
    c2.wait()
    c3.wait()
    c4.wait()

    lane = lax.iota(jnp.int32, _L)

    @plsc.parallel_loop(0, _EPW // _L, 1, unroll=25)
    def _(i):
        off = off0 + i * _L
        d16 = dst_v[pl.ds(off, _L)]
        s16 = src_v[pl.ds(off, _L)]
        q16 = q_v[pl.ds(off, _L)]
        g16 = plsc.load_gather(batch_v, [d16])
        p16 = plsc.load_gather(p_v, [s16])
        idx = g16 * _L + lane
        plsc.addupdate_scatter(acc_v, [idx], p16 + q16)

    pltpu.sync_copy(acc_v, out_hbm.at[wid])


def _sc_edges(src2, dst2, q2, batch, p2):
    mesh = plsc.VectorSubcoreMesh(core_axis_name="c", subcore_axis_name="s")
    run = pl.kernel(
        _sc_edges_body, mesh=mesh,
        compiler_params=pltpu.CompilerParams(needs_layout_passes=False),
        out_type=jax.ShapeDtypeStruct((_NW, _G * _L), jnp.float32),
        scratch_types=[
            pltpu.VMEM((_CP,), jnp.int32),
            pltpu.VMEM((_CP,), jnp.int32),
            pltpu.VMEM((_CP,), jnp.float32),
            pltpu.VMEM((_N,), jnp.int32),
            pltpu.VMEM((_N,), jnp.float32),
            pltpu.VMEM((_G * _L,), jnp.float32),
            pltpu.SemaphoreType.DMA,
            pltpu.SemaphoreType.DMA,
            pltpu.SemaphoreType.DMA,
            pltpu.SemaphoreType.DMA,
            pltpu.SemaphoreType.DMA,
        ],
    )
    return run(src2, dst2, q2, batch, p2)


def _finish_body(part_ref, w3_ref, b2_ref, b3_ref, o_ref):
    colsum = jnp.sum(part_ref[...], axis=0, keepdims=True)
    m = lax.broadcasted_iota(jnp.int32, (_G * _L, _G), 0)
    c = lax.broadcasted_iota(jnp.int32, (_G * _L, _G), 1)
    sel = jnp.where(m // _L == c, 1.0, 0.0).astype(jnp.float32)
    s2 = jnp.dot(colsum, sel, preferred_element_type=jnp.float32,
                 precision=lax.Precision.HIGHEST)
    c0 = jnp.sum(w3_ref[...] * b2_ref[...]) + b3_ref[0, 0]
    o_ref[...] = s2 + c0


def _finish(part, w3, b2r, b3r):
    return pl.pallas_call(
        _finish_body,
        out_shape=jax.ShapeDtypeStruct((1, _G), jnp.float32),
    )(part, w3, b2r, b3r)


def kernel(x, edge_index, edge_attr, smiles, batch, W1, b1, W2, b2, W3, b3):
    p, q, src2, dst2 = _prep(x, edge_attr.T, edge_index, W1,
                             b1.reshape(1, -1), W2, W3)
    part = _sc_edges(src2, dst2, q, batch, p)
    out = _finish(part, W3, b2.reshape(1, -1), b3.reshape(1, 1))
    return out.reshape(_G, 1)
